# baseline (speedup 1.0000x reference)
import functools

import jax
import jax.numpy as jnp
from jax import lax
from jax.experimental import pallas as pl
from jax.experimental.pallas import tpu as pltpu
from jax.experimental.pallas import tpu_sc as plsc

DIM = 64
NC = 2
NS = 16
NW = NC * NS
CHUNK = 256


NB = 2


@functools.lru_cache(maxsize=None)
def _make_lookup(n_chunks):
  assert n_chunks % NB == 0
  n_groups = n_chunks // NB
  mesh = plsc.VectorSubcoreMesh(core_axis_name="c", subcore_axis_name="s")

  @functools.partial(
      pl.kernel,
      mesh=mesh,
      compiler_params=pltpu.CompilerParams(use_tc_tiling_on_sc=False),
      out_type=jax.ShapeDtypeStruct((NW, n_chunks, CHUNK, DIM), jnp.float32),
      scratch_types=[
          pltpu.VMEM((n_chunks, CHUNK), jnp.int32),
          pltpu.VMEM((n_chunks, CHUNK), jnp.int32),
          pltpu.VMEM((NB, CHUNK, DIM), jnp.float32),
          pltpu.VMEM_SHARED((2049, DIM), jnp.float32),
          pltpu.VMEM_SHARED((2049, DIM), jnp.float32),
      ] + [pltpu.SemaphoreType.DMA] * NB,
  )
  def lookup(t_tab, s_tab, t_idx, s_idx, out, tiv, siv, bufs, t_sh, s_sh,
             *sems):
    sid = lax.axis_index("s")
    wid = sid * NC + lax.axis_index("c")

    @pl.when(sid == 0)
    def _():
      pltpu.sync_copy(t_tab, t_sh)
      pltpu.sync_copy(s_tab, s_sh)

    pltpu.sync_copy(t_idx.at[wid], tiv)
    pltpu.sync_copy(s_idx.at[wid], siv)
    plsc.subcore_barrier()

    def fire_t(c, b):
      pltpu.async_copy(t_sh.at[tiv.at[c]], bufs.at[b], sems[b])

    def wait_t(c, b):
      pltpu.make_async_copy(t_sh.at[tiv.at[c]], bufs.at[b], sems[b]).wait()

    def fire_s(c, b):
      pltpu.async_copy(s_sh.at[siv.at[c]], bufs.at[b], sems[b], add=True)

    def wait_s(c, b):
      pltpu.make_async_copy(s_sh.at[siv.at[c]], bufs.at[b], sems[b]).wait()

    def fire_out(c, b):
      pltpu.async_copy(bufs.at[b], out.at[wid, c], sems[b])

    def wait_out(c, b):
      pltpu.make_async_copy(bufs.at[b], out.at[wid, c], sems[b]).wait()

    for b in range(NB):
      fire_t(b, b)

    def group_body(g, carry):
      base = g * NB
      for b in range(NB):
        wait_t(base + b, b)
        fire_s(base + b, b)
      for b in range(NB):
        wait_s(base + b, b)
        fire_out(base + b, b)
      for b in range(NB):
        wait_out(base + b, b)

        @pl.when(g < n_groups - 1)
        def _():
          fire_t(base + NB + b, b)

      return carry

    lax.fori_loop(0, n_groups, group_body, 0)

  return lookup


def kernel(times, spaces, time_emb, space_emb):
  B, L = times.shape
  n = B * L
  assert n % (NW * CHUNK) == 0
  n_chunks = n // (NW * CHUNK)
  t_idx = times.reshape(NW, n_chunks, CHUNK).astype(jnp.int32)
  s_idx = spaces.reshape(NW, n_chunks, CHUNK).astype(jnp.int32)
  out = _make_lookup(n_chunks)(time_emb, space_emb, t_idx, s_idx)
  return out.reshape(B, L, DIM)


# speedup vs baseline: 1.0166x; 1.0166x over previous
import functools

import jax
import jax.numpy as jnp
from jax import lax
from jax.experimental import pallas as pl
from jax.experimental.pallas import tpu as pltpu
from jax.experimental.pallas import tpu_sc as plsc

DIM = 64
NC = 2
NS = 16
NW = NC * NS
NB = 4


@functools.lru_cache(maxsize=None)
def _make_lookup(B, L, n_rows):
  n_chunks = B // NW
  assert n_chunks % NB == 0
  n_groups = n_chunks // NB
  mesh = plsc.VectorSubcoreMesh(core_axis_name="c", subcore_axis_name="s")

  @functools.partial(
      pl.kernel,
      mesh=mesh,
      compiler_params=pltpu.CompilerParams(use_tc_tiling_on_sc=False),
      out_type=jax.ShapeDtypeStruct((B, L, DIM), jnp.float32),
      scratch_types=[
          pltpu.VMEM((n_chunks, L), jnp.int32),
          pltpu.VMEM((n_chunks, L), jnp.int32),
          pltpu.VMEM((NB, L, DIM), jnp.float32),
          pltpu.VMEM_SHARED((n_rows, DIM), jnp.float32),
          pltpu.VMEM_SHARED((n_rows, DIM), jnp.float32),
      ] + [pltpu.SemaphoreType.DMA] * NB,
  )
  def lookup(t_tab, s_tab, t_idx, s_idx, out, tiv, siv, bufs, t_sh, s_sh,
             *sems):
    sid = lax.axis_index("s")
    wid = sid * NC + lax.axis_index("c")

    @pl.when(sid == 0)
    def _():
      pltpu.sync_copy(t_tab, t_sh)
      pltpu.sync_copy(s_tab, s_sh)

    pltpu.sync_copy(t_idx.at[wid], tiv)
    pltpu.sync_copy(s_idx.at[wid], siv)
    plsc.subcore_barrier()

    def fire_t(c, b):
      pltpu.async_copy(t_sh.at[tiv.at[c]], bufs.at[b], sems[b])

    def wait_t(c, b):
      pltpu.make_async_copy(t_sh.at[tiv.at[c]], bufs.at[b], sems[b]).wait()

    def fire_s(c, b):
      pltpu.async_copy(s_sh.at[siv.at[c]], bufs.at[b], sems[b], add=True)

    def wait_s(c, b):
      pltpu.make_async_copy(s_sh.at[siv.at[c]], bufs.at[b], sems[b]).wait()

    def fire_out(c, b):
      pltpu.async_copy(bufs.at[b], out.at[wid * n_chunks + c], sems[b])

    def wait_out(c, b):
      pltpu.make_async_copy(bufs.at[b], out.at[wid * n_chunks + c],
                            sems[b]).wait()

    for b in range(NB):
      fire_t(b, b)

    def group_body(g, carry):
      base = g * NB
      for b in range(NB):
        wait_t(base + b, b)
        fire_s(base + b, b)
      for b in range(NB):
        wait_s(base + b, b)
        fire_out(base + b, b)
      for b in range(NB):
        wait_out(base + b, b)

        @pl.when(g < n_groups - 1)
        def _():
          fire_t(base + NB + b, b)

      return carry

    lax.fori_loop(0, n_groups, group_body, 0)

  return lookup


def kernel(times, spaces, time_emb, space_emb):
  B, L = times.shape
  assert B % NW == 0
  n_chunks = B // NW
  t_idx = times.reshape(NW, n_chunks, L).astype(jnp.int32)
  s_idx = spaces.reshape(NW, n_chunks, L).astype(jnp.int32)
  return _make_lookup(B, L, time_emb.shape[0])(time_emb, space_emb, t_idx,
                                               s_idx)
